# baseline (speedup 1.0000x reference)
import functools

import jax
import jax.numpy as jnp
from jax import lax
from jax.experimental import pallas as pl
from jax.experimental.pallas import tpu as pltpu
from jax.experimental.pallas import tpu_sc as plsc

_NUM_EMBEDDINGS = 1000000
_DIM = 64
_BATCH = 16384

_info = plsc.get_sparse_core_info()
_NC, _NS = _info.num_cores, _info.num_subcores
_NW = _NC * _NS
_B_PER_W = _BATCH // _NW

_mesh = plsc.VectorSubcoreMesh(core_axis_name="c", subcore_axis_name="s")


@functools.partial(
    pl.kernel,
    mesh=_mesh,
    compiler_params=pltpu.CompilerParams(use_tc_tiling_on_sc=False),
    out_type=jax.ShapeDtypeStruct((_BATCH, _DIM), jnp.float32),
    scratch_types=[
        pltpu.VMEM((_B_PER_W,), jnp.int32),
        pltpu.VMEM((_B_PER_W, _DIM), jnp.float32),
        pltpu.SemaphoreType.DMA,
    ],
)
def _gather_kernel(idx_hbm, table_hbm, out_hbm, idx_v, rows_v, sem):
    wid = lax.axis_index("s") * _NC + lax.axis_index("c")
    base = wid * _B_PER_W
    pltpu.sync_copy(idx_hbm.at[pl.ds(base, _B_PER_W)], idx_v)
    pltpu.async_copy(table_hbm.at[idx_v], rows_v, sem).wait()
    pltpu.sync_copy(rows_v, out_hbm.at[pl.ds(base, _B_PER_W)])


@jax.jit
def kernel(x, table):
    idx = x.reshape(-1)
    return _gather_kernel(idx, table)


# speedup vs baseline: 2.3167x; 2.3167x over previous
import functools

import jax
import jax.numpy as jnp
from jax import lax
from jax.experimental import pallas as pl
from jax.experimental.pallas import tpu as pltpu
from jax.experimental.pallas import tpu_sc as plsc

_NUM_EMBEDDINGS = 1000000
_DIM = 64
_BATCH = 16384
_LANES = 128

_info = plsc.get_sparse_core_info()
_NC, _NS = _info.num_cores, _info.num_subcores
_NW = _NC * _NS
_B_PER_W = _BATCH // _NW
_CHUNK = 8
_FLUSH = 128
_N_CHUNKS_PER_FLUSH = _FLUSH // _CHUNK
_N_FLUSH = _B_PER_W // _FLUSH

_mesh = plsc.VectorSubcoreMesh(core_axis_name="c", subcore_axis_name="s")


@functools.partial(
    pl.kernel,
    mesh=_mesh,
    compiler_params=pltpu.CompilerParams(needs_layout_passes=False),
    out_type=jax.ShapeDtypeStruct((_DIM, _BATCH), jnp.float32),
    scratch_types=[
        pltpu.VMEM((_B_PER_W + 16,), jnp.int32),
        pltpu.VMEM((_CHUNK, _DIM, _LANES), jnp.float32),
        pltpu.VMEM((_DIM, _FLUSH), jnp.float32),
        pltpu.SemaphoreType.DMA,
    ],
)
def _gather_kernel(idx_hbm, tt_hbm, out_hbm, idx_v, ring, staging, sem):
    wid = lax.axis_index("s") * _NC + lax.axis_index("c")
    base = pl.multiple_of(wid * _B_PER_W, _B_PER_W)
    pltpu.sync_copy(idx_hbm.at[pl.ds(base, _B_PER_W)], idx_v.at[pl.ds(0, _B_PER_W)])

    lane = lax.iota(jnp.int32, 16)
    rows = [lane + 16 * v for v in range(4)]

    for blk in range(_N_FLUSH):

        def chunk_body(c, blk=blk):
            cbase = blk * _FLUSH + c * _CHUNK
            vec = idx_v[pl.ds(cbase, 16)]
            ks = [
                jnp.sum(jnp.where(lane == j, vec, 0)) for j in range(_CHUNK)
            ]
            copies = []
            for j in range(_CHUNK):
                k128 = pl.multiple_of((ks[j] // _LANES) * _LANES, _LANES)
                copies.append(
                    pltpu.async_copy(
                        tt_hbm.at[:, pl.ds(k128, _LANES)], ring.at[j], sem
                    )
                )
            for cp in copies:
                cp.wait()
            for j in range(_CHUNK):
                l_vec = jnp.broadcast_to(ks[j] % _LANES, (16,))
                c_vec = jnp.broadcast_to((c * _CHUNK + j) % _FLUSH, (16,))
                for v in range(4):
                    vals = plsc.load_gather(ring.at[j], [rows[v], l_vec])
                    plsc.store_scatter(staging, [rows[v], c_vec], vals)

        pl.loop(0, _N_CHUNKS_PER_FLUSH)(chunk_body)
        out_off = pl.multiple_of(base + blk * _FLUSH, _FLUSH)
        pltpu.sync_copy(staging, out_hbm.at[:, pl.ds(out_off, _FLUSH)])


@jax.jit
def kernel(x, table):
    idx = x.reshape(-1)
    out_t = _gather_kernel(idx, table.T)
    return out_t.T


# speedup vs baseline: 3.1267x; 1.3496x over previous
import functools

import jax
import jax.numpy as jnp
from jax import lax
from jax.experimental import pallas as pl
from jax.experimental.pallas import tpu as pltpu
from jax.experimental.pallas import tpu_sc as plsc

_N_EMB = 1000000
_DIM = 64
_BATCH = 16384
_LANES = 128
_NBLK = (_N_EMB + _LANES - 1) // _LANES

_info = plsc.get_sparse_core_info()
_NC, _NS = _info.num_cores, _info.num_subcores
_NW = _NC * _NS
_BPW = (_NBLK + _NW - 1) // _NW
_CAP = _BATCH
_RING = 4
_OCH = 64
_NCH = _CAP // _OCH
_DUMP = _BATCH

_mesh = plsc.VectorSubcoreMesh(core_axis_name="c", subcore_axis_name="s")


@functools.partial(
    pl.kernel,
    mesh=_mesh,
    compiler_params=pltpu.CompilerParams(needs_layout_passes=False),
    out_type=jax.ShapeDtypeStruct((_BATCH + _OCH, _LANES), jnp.float32),
    scratch_types=[
        pltpu.VMEM((2048,), jnp.int32),
        pltpu.VMEM((_CAP + 16,), jnp.int32),
        pltpu.VMEM((_CAP + 16,), jnp.int32),
        pltpu.VMEM((_CAP + 16,), jnp.int32),
        pltpu.VMEM((_NCH + 1, _OCH), jnp.int32),
        pltpu.VMEM((272,), jnp.int32),
        pltpu.VMEM((272,), jnp.int32),
        pltpu.VMEM((272,), jnp.int32),
        pltpu.VMEM((272,), jnp.int32),
        pltpu.VMEM((_RING, _DIM, _LANES), jnp.float32),
        pltpu.VMEM((_OCH, _LANES), jnp.float32),
        pltpu.SemaphoreType.DMA,
    ],
)
def _gather_kernel(
    idx_hbm, tt_hbm, out_hbm,
    idx_buf, mk, mb, sk, sb, hist, starts, offs, blist, ring, ostage, sem,
):
    wid = lax.axis_index("s") * _NC + lax.axis_index("c")
    c0 = wid * _BPW
    lane = lax.iota(jnp.int32, 16)

    def ext(ref, pos):
        vec = ref[pl.ds((pos >> 4) << 4, 16)]
        return jnp.sum(jnp.where(lane == (pos & 15), vec, 0))

    count = jnp.int32(0)
    for oc in range(_BATCH // 2048):
        pltpu.sync_copy(idx_hbm.at[pl.ds(oc * 2048, 2048)], idx_buf)

        def abody(i, cnt, oc=oc):
            vec = idx_buf[pl.ds(i * 16, 16)]
            crel = (vec >> 7) - c0
            mask = (crel >= 0) & (crel < _BPW)
            mi = mask.astype(jnp.int32)
            pos = cnt + plsc.cumsum(mi) - mi
            posm = jnp.where(mask, pos, _CAP)
            plsc.store_scatter(mk, [posm], vec)
            plsc.store_scatter(mb, [posm], oc * 2048 + i * 16 + lane)
            return cnt + jnp.sum(mi)

        count = lax.fori_loop(0, 2048 // 16, abody, count)

    zeros16 = jnp.zeros((16,), jnp.int32)
    for t in range(16):
        hist[pl.ds(t * 16, 16)] = zeros16

    def b1(j, _):
        cb = jnp.broadcast_to((ext(mk, j) >> 7) - c0, (16,))
        h = plsc.load_gather(hist, [cb])
        plsc.store_scatter(hist, [cb], h + 1)
        return 0

    lax.fori_loop(0, count, b1, 0)

    carry = jnp.int32(0)
    for t in range(16):
        h = hist[pl.ds(t * 16, 16)]
        excl = plsc.cumsum(h) - h + carry
        starts[pl.ds(t * 16, 16)] = excl
        offs[pl.ds(t * 16, 16)] = excl
        carry = carry + jnp.sum(h)

    def b3(j, _):
        kj = ext(mk, j)
        bj = ext(mb, j)
        cb = jnp.broadcast_to((kj >> 7) - c0, (16,))
        o = plsc.load_gather(offs, [cb])
        plsc.store_scatter(offs, [cb], o + 1)
        plsc.store_scatter(sk, [o], jnp.broadcast_to(kj, (16,)))
        plsc.store_scatter(
            sb, [o >> 6, o & (_OCH - 1)], jnp.broadcast_to(bj, (16,))
        )
        return 0

    lax.fori_loop(0, count, b3, 0)

    nb = jnp.int32(0)
    for t in range(16):
        h = hist[pl.ds(t * 16, 16)]
        m = (h > 0).astype(jnp.int32)
        pos = nb + plsc.cumsum(m) - m
        posm = jnp.where(h > 0, pos, 256)
        plsc.store_scatter(blist, [posm], t * 16 + lane)
        nb = nb + jnp.sum(m)

    def fetch(slot, bi):
        col = pl.multiple_of((ext(blist, bi) + c0) * _LANES, _LANES)
        pltpu.async_copy(tt_hbm.at[:, pl.ds(col, _LANES)], ring.at[slot], sem)

    for s in range(_RING):

        @pl.when(s < nb)
        def _(s=s):
            fetch(s, s)

    def proc(bi, j):
        slot = bi % _RING
        pltpu.make_async_copy(
            tt_hbm.at[:, pl.ds(0, _LANES)], ring.at[slot], sem
        ).wait()
        c_rel = ext(blist, bi)
        base_k = (c_rel + c0) * _LANES
        e_c = ext(starts, c_rel + 1)

        def mbody(j2, _):
            lvec = jnp.broadcast_to(ext(sk, j2) - base_k, (16,))
            bufrow = j2 & (_OCH - 1)
            for v in range(4):
                vals = plsc.load_gather(ring.at[slot], [lane + 16 * v, lvec])
                ostage[bufrow, pl.ds(16 * v, 16)] = vals

            @pl.when(bufrow == _OCH - 1)
            def _():
                pltpu.sync_copy(ostage, out_hbm.at[sb.at[j2 >> 6]])

            return 0

        lax.fori_loop(j, e_c, mbody, 0)

        @pl.when(bi + _RING < nb)
        def _():
            fetch(slot, bi + _RING)

        return e_c

    count_end = lax.fori_loop(0, nb, proc, jnp.int32(0))

    rem = count_end & (_OCH - 1)

    @pl.when(rem > 0)
    def _():
        q = count_end >> 6
        for g in range(4):
            cur = sb[q, pl.ds(g * 16, 16)]
            msk = (g * 16 + lane) < rem
            sb[q, pl.ds(g * 16, 16)] = jnp.where(msk, cur, _DUMP)
        pltpu.sync_copy(ostage, out_hbm.at[sb.at[q]])


@jax.jit
def kernel(x, table):
    idx = x.reshape(-1)
    out_pad = _gather_kernel(idx, table.T)
    return out_pad[:_BATCH, :_DIM]


# speedup vs baseline: 3.5574x; 1.1378x over previous
import functools

import jax
import jax.numpy as jnp
from jax import lax
from jax.experimental import pallas as pl
from jax.experimental.pallas import tpu as pltpu
from jax.experimental.pallas import tpu_sc as plsc

_N_EMB = 1000000
_DIM = 64
_BATCH = 16384
_LANES = 128
_NBLK = (_N_EMB + _LANES - 1) // _LANES

_info = plsc.get_sparse_core_info()
_NC, _NS = _info.num_cores, _info.num_subcores
_NW = _NC * _NS
_BPW = (_NBLK + _NW - 1) // _NW
_CAP = _BATCH
_RING = 4
_OCH = 64
_NCH = _CAP // _OCH
_DUMP = _BATCH

_mesh = plsc.VectorSubcoreMesh(core_axis_name="c", subcore_axis_name="s")


@functools.partial(
    pl.kernel,
    mesh=_mesh,
    compiler_params=pltpu.CompilerParams(needs_layout_passes=False),
    out_type=jax.ShapeDtypeStruct((_BATCH + _OCH, _LANES), jnp.float32),
    scratch_types=[
        pltpu.VMEM((2048,), jnp.int32),
        pltpu.VMEM((_CAP + 16,), jnp.int32),
        pltpu.VMEM((_CAP + 16,), jnp.int32),
        pltpu.VMEM((_CAP + 16,), jnp.int32),
        pltpu.VMEM((_NCH + 1, _OCH), jnp.int32),
        pltpu.VMEM((272,), jnp.int32),
        pltpu.VMEM((272,), jnp.int32),
        pltpu.VMEM((272,), jnp.int32),
        pltpu.VMEM((272,), jnp.int32),
        pltpu.VMEM((_RING, _DIM, _LANES), jnp.float32),
        pltpu.VMEM((_OCH, _LANES), jnp.float32),
        pltpu.SemaphoreType.DMA,
    ],
)
def _gather_kernel(
    idx_hbm, tt_hbm, out_hbm,
    idx_buf, mk, mb, sk, sb, hist, starts, offs, blist, ring, ostage, sem,
):
    wid = lax.axis_index("s") * _NC + lax.axis_index("c")
    c0 = wid * _BPW
    lane = lax.iota(jnp.int32, 16)

    def ext(ref, pos):
        vec = ref[pl.ds((pos >> 4) << 4, 16)]
        return jnp.sum(jnp.where(lane == (pos & 15), vec, 0))

    zeros16 = jnp.zeros((16,), jnp.int32)
    for t in range(16):
        hist[pl.ds(t * 16, 16)] = zeros16

    count = jnp.int32(0)
    for oc in range(_BATCH // 2048):
        pltpu.sync_copy(idx_hbm.at[pl.ds(oc * 2048, 2048)], idx_buf)

        def abody(i, cnt, oc=oc):
            vec = idx_buf[pl.ds(i * 16, 16)]
            crel = (vec >> 7) - c0
            mask = (crel >= 0) & (crel < _BPW)
            mi = mask.astype(jnp.int32)
            pos = cnt + plsc.cumsum(mi) - mi
            plsc.store_scatter(mk, [pos], vec, mask=mask)
            plsc.store_scatter(mb, [pos], oc * 2048 + i * 16 + lane, mask=mask)
            plsc.addupdate_scatter(hist, [crel], mi, mask=mask)
            return cnt + jnp.sum(mi)

        count = plsc.parallel_loop(0, 2048 // 16, unroll=4, carry=count)(abody)

    carry = jnp.int32(0)
    for t in range(16):
        h = hist[pl.ds(t * 16, 16)]
        excl = plsc.cumsum(h) - h + carry
        starts[pl.ds(t * 16, 16)] = excl
        offs[pl.ds(t * 16, 16)] = excl
        carry = carry + jnp.sum(h)

    def b3(j, _):
        kj = ext(mk, j)
        bj = ext(mb, j)
        cb = jnp.broadcast_to((kj >> 7) - c0, (16,))
        o = plsc.load_gather(offs, [cb])
        plsc.store_scatter(offs, [cb], o + 1)
        plsc.store_scatter(sk, [o], jnp.broadcast_to(kj, (16,)))
        plsc.store_scatter(
            sb, [o >> 6, o & (_OCH - 1)], jnp.broadcast_to(bj, (16,))
        )
        return 0

    lax.fori_loop(0, count, b3, 0)

    nb = jnp.int32(0)
    for t in range(16):
        h = hist[pl.ds(t * 16, 16)]
        m = (h > 0).astype(jnp.int32)
        pos = nb + plsc.cumsum(m) - m
        posm = jnp.where(h > 0, pos, 256)
        plsc.store_scatter(blist, [posm], t * 16 + lane)
        nb = nb + jnp.sum(m)

    def fetch(slot, bi):
        col = pl.multiple_of((ext(blist, bi) + c0) * _LANES, _LANES)
        pltpu.async_copy(tt_hbm.at[:, pl.ds(col, _LANES)], ring.at[slot], sem)

    for s in range(_RING):

        @pl.when(s < nb)
        def _(s=s):
            fetch(s, s)

    def proc(bi, j):
        slot = bi % _RING
        pltpu.make_async_copy(
            tt_hbm.at[:, pl.ds(0, _LANES)], ring.at[slot], sem
        ).wait()
        c_rel = ext(blist, bi)
        base_k = (c_rel + c0) * _LANES
        e_c = ext(starts, c_rel + 1)

        def mbody(j2, _):
            lvec = jnp.broadcast_to(ext(sk, j2) - base_k, (16,))
            bufrow = j2 & (_OCH - 1)
            for v in range(4):
                vals = plsc.load_gather(ring.at[slot], [lane + 16 * v, lvec])
                ostage[bufrow, pl.ds(16 * v, 16)] = vals

            @pl.when(bufrow == _OCH - 1)
            def _():
                pltpu.sync_copy(ostage, out_hbm.at[sb.at[j2 >> 6]])

            return 0

        lax.fori_loop(j, e_c, mbody, 0)

        @pl.when(bi + _RING < nb)
        def _():
            fetch(slot, bi + _RING)

        return e_c

    count_end = lax.fori_loop(0, nb, proc, jnp.int32(0))

    rem = count_end & (_OCH - 1)

    @pl.when(rem > 0)
    def _():
        q = count_end >> 6
        for g in range(4):
            cur = sb[q, pl.ds(g * 16, 16)]
            msk = (g * 16 + lane) < rem
            sb[q, pl.ds(g * 16, 16)] = jnp.where(msk, cur, _DUMP)
        pltpu.sync_copy(ostage, out_hbm.at[sb.at[q]])


@jax.jit
def kernel(x, table):
    idx = x.reshape(-1)
    out_pad = _gather_kernel(idx, table.T)
    return out_pad[:_BATCH, :_DIM]


# speedup vs baseline: 3.7838x; 1.0637x over previous
import functools

import jax
import jax.numpy as jnp
from jax import lax
from jax.experimental import pallas as pl
from jax.experimental.pallas import tpu as pltpu
from jax.experimental.pallas import tpu_sc as plsc

_N_EMB = 1000000
_DIM = 64
_BATCH = 16384
_LANES = 128
_NBLK = (_N_EMB + _LANES - 1) // _LANES

_info = plsc.get_sparse_core_info()
_NC, _NS = _info.num_cores, _info.num_subcores
_NW = _NC * _NS
_BPW = (_NBLK + _NW - 1) // _NW
_CAP = _BATCH
_RING = 8
_OCH = 64
_NCH = _CAP // _OCH
_DUMP = _BATCH

_mesh = plsc.VectorSubcoreMesh(core_axis_name="c", subcore_axis_name="s")


@functools.partial(
    pl.kernel,
    mesh=_mesh,
    compiler_params=pltpu.CompilerParams(needs_layout_passes=False),
    out_type=jax.ShapeDtypeStruct((_BATCH + _OCH, _LANES), jnp.float32),
    scratch_types=[
        pltpu.VMEM((2048,), jnp.int32),
        pltpu.VMEM((_CAP + 16,), jnp.int32),
        pltpu.VMEM((_NCH + 1, _OCH), jnp.int32),
        pltpu.VMEM((272,), jnp.int32),
        pltpu.VMEM((272,), jnp.int32),
        pltpu.VMEM((272,), jnp.int32),
        pltpu.VMEM((272,), jnp.int32),
        pltpu.VMEM((_RING, _DIM, _LANES), jnp.float32),
        pltpu.VMEM((_OCH, _LANES), jnp.float32),
        pltpu.SemaphoreType.DMA,
    ],
)
def _gather_kernel(
    idx_hbm, tt_hbm, out_hbm,
    idx_buf, mkb, sb, hist, starts, offs, blist, ring, ostage, sem,
):
    wid = lax.axis_index("s") * _NC + lax.axis_index("c")
    c0 = wid * _BPW
    lane = lax.iota(jnp.int32, 16)

    def ext(ref, pos):
        vec = ref[pl.ds((pos >> 4) << 4, 16)]
        return jnp.sum(jnp.where(lane == (pos & 15), vec, 0))

    zeros16 = jnp.zeros((16,), jnp.int32)
    for t in range(16):
        hist[pl.ds(t * 16, 16)] = zeros16

    count = jnp.int32(0)
    for oc in range(_BATCH // 2048):
        pltpu.sync_copy(idx_hbm.at[pl.ds(oc * 2048, 2048)], idx_buf)

        def abody(i, cnt, oc=oc):
            vec = idx_buf[pl.ds(i * 16, 16)]
            crel = (vec >> 7) - c0
            mask = (crel >= 0) & (crel < _BPW)
            mi = mask.astype(jnp.int32)
            pos = cnt + plsc.cumsum(mi) - mi
            packed = (
                (oc * 2048 + i * 16 + lane)
                | ((vec & (_LANES - 1)) << 14)
                | (crel << 21)
            )
            plsc.store_scatter(mkb, [pos], packed, mask=mask)
            plsc.addupdate_scatter(hist, [crel], mi, mask=mask)
            return cnt + jnp.sum(mi)

        count = plsc.parallel_loop(0, 2048 // 16, unroll=4, carry=count)(abody)

    carry = jnp.int32(0)
    for t in range(16):
        h = hist[pl.ds(t * 16, 16)]
        excl = plsc.cumsum(h) - h + carry
        starts[pl.ds(t * 16, 16)] = excl
        offs[pl.ds(t * 16, 16)] = excl
        carry = carry + jnp.sum(h)

    def b3(j, _):
        pj = ext(mkb, j)
        cb = jnp.broadcast_to(pj >> 21, (16,))
        o = plsc.load_gather(offs, [cb])
        plsc.store_scatter(offs, [cb], o + 1)
        plsc.store_scatter(
            sb,
            [o >> 6, o & (_OCH - 1)],
            jnp.broadcast_to(pj & ((1 << 21) - 1), (16,)),
        )
        return 0

    lax.fori_loop(0, count, b3, 0)

    nb = jnp.int32(0)
    for t in range(16):
        h = hist[pl.ds(t * 16, 16)]
        m = (h > 0).astype(jnp.int32)
        pos = nb + plsc.cumsum(m) - m
        posm = jnp.where(h > 0, pos, 256)
        plsc.store_scatter(blist, [posm], t * 16 + lane)
        nb = nb + jnp.sum(m)

    def fetch(slot, bi):
        col = pl.multiple_of((ext(blist, bi) + c0) * _LANES, _LANES)
        pltpu.async_copy(tt_hbm.at[:, pl.ds(col, _LANES)], ring.at[slot], sem)

    for s in range(_RING):

        @pl.when(s < nb)
        def _(s=s):
            fetch(s, s)

    def proc(bi, j):
        slot = bi % _RING
        pltpu.make_async_copy(
            tt_hbm.at[:, pl.ds(0, _LANES)], ring.at[slot], sem
        ).wait()
        c_rel = ext(blist, bi)
        e_c = ext(starts, c_rel + 1)

        def mbody(j2, _):
            q = j2 >> 6
            ln = j2 & (_OCH - 1)
            vec = sb[q, pl.ds((ln >> 4) << 4, 16)]
            pv = jnp.sum(jnp.where(lane == (ln & 15), vec, 0))
            lvec = jnp.broadcast_to(pv >> 14, (16,))
            for v in range(4):
                vals = plsc.load_gather(ring.at[slot], [lane + 16 * v, lvec])
                ostage[ln, pl.ds(16 * v, 16)] = vals

            @pl.when(ln == _OCH - 1)
            def _():
                for g in range(4):
                    cur = sb[q, pl.ds(g * 16, 16)]
                    sb[q, pl.ds(g * 16, 16)] = cur & 16383
                pltpu.sync_copy(ostage, out_hbm.at[sb.at[q]])

            return 0

        lax.fori_loop(j, e_c, mbody, 0)

        @pl.when(bi + _RING < nb)
        def _():
            fetch(slot, bi + _RING)

        return e_c

    count_end = lax.fori_loop(0, nb, proc, jnp.int32(0))

    rem = count_end & (_OCH - 1)

    @pl.when(rem > 0)
    def _():
        q = count_end >> 6
        for g in range(4):
            cur = sb[q, pl.ds(g * 16, 16)]
            msk = (g * 16 + lane) < rem
            sb[q, pl.ds(g * 16, 16)] = jnp.where(msk, cur & 16383, _DUMP)
        pltpu.sync_copy(ostage, out_hbm.at[sb.at[q]])


@jax.jit
def kernel(x, table):
    idx = x.reshape(-1)
    out_pad = _gather_kernel(idx, table.T)
    return out_pad[:_BATCH, :_DIM]
